# baseline (speedup 1.0000x reference)
import functools

import jax
import jax.numpy as jnp
from jax import lax
from jax.experimental import pallas as pl
from jax.experimental.pallas import tpu as pltpu
from jax.experimental.pallas import tpu_sc as plsc

NUM_ENT = 1000000
EMBED_DIM = 64
BATCH = 16384

NC = 2
NS = 16
L = 16
NW = NC * NS
B_PER_W = BATCH // NW
QB = 128
NQ = B_PER_W // QB

TBLK = 65536
HT = TBLK // 2
SHB = TBLK.bit_length() - 1
NBLK = -(-NUM_ENT // TBLK)
QT = TBLK // 4
YROWS = NBLK * QT


def _tc_pack_body(x_ref, y_ref):
    xi = lax.bitcast_convert_type(x_ref[...], jnp.int32)
    lo = jnp.right_shift(xi[0:32, :] + 0x8000, 16) & jnp.int32(0xFFFF)
    hi = (xi[32:64, :] + 0x8000) & jnp.int32(-65536)
    w = lax.bitcast_convert_type(hi | lo, jnp.float32)
    v = jnp.concatenate(
        [w[:, m * QT:(m + 1) * QT] for m in range(4)], axis=0)
    y_ref[...] = v.T


_tc_pack = pl.pallas_call(
    _tc_pack_body,
    grid=(NBLK,),
    in_specs=[pl.BlockSpec((EMBED_DIM, TBLK), lambda i: (0, i))],
    out_specs=pl.BlockSpec((QT, 2 * EMBED_DIM), lambda i: (i, 0)),
    out_shape=jax.ShapeDtypeStruct((YROWS, 2 * EMBED_DIM), jnp.float32),
    compiler_params=pltpu.CompilerParams(
        dimension_semantics=("arbitrary",)),
)


def _norm_neg_sqrt(acc):
    xs = jnp.maximum(acc, 1e-20)
    bits = lax.bitcast_convert_type(xs, jnp.int32)
    bits = jnp.int32(0x5F3759DF) - jnp.right_shift(bits, 1)
    y = lax.bitcast_convert_type(bits, jnp.float32)
    y = y * (1.5 - 0.5 * xs * y * y)
    y = y * (1.5 - 0.5 * xs * y * y)
    return -(xs * y)


def _sc_body(y, h_idx, t_idx, out,
             raw_h, raw_t, row_h, row_t, half_h, half_t,
             hbuf, tbuf, out_v, sems):
    wid = lax.axis_index("s") * NC + lax.axis_index("c")
    base = wid * B_PER_W

    pltpu.sync_copy(h_idx.at[pl.ds(base, B_PER_W)], raw_h)
    pltpu.sync_copy(t_idx.at[pl.ds(base, B_PER_W)], raw_t)

    def prep_body(k, carry):
        off = pl.multiple_of(k * L, L)
        for raw, row, half in ((raw_h, row_h, half_h), (raw_t, row_t, half_t)):
            e = raw[pl.ds(off, L)]
            row[pl.ds(off, L)] = (jnp.left_shift(jnp.right_shift(e, SHB), SHB - 2)
                                  | (e & (QT - 1)))
            half[pl.ds(off, L)] = jnp.left_shift(jnp.right_shift(e, SHB - 2) & 3, 5)
        return carry

    lax.fori_loop(0, B_PER_W // L, prep_body, 0)

    def fire(q):
        return (
            pltpu.async_copy(y.at[row_h.at[pl.ds(q * QB, QB)]],
                             hbuf[q % 2], sems[q % 2]),
            pltpu.async_copy(y.at[row_t.at[pl.ds(q * QB, QB)]],
                             tbuf[q % 2], sems[q % 2]),
        )

    lanes = lax.iota(jnp.int32, L)
    descs = {0: fire(0)}
    for q in range(NQ):
        if q + 1 < NQ:
            descs[q + 1] = fire(q + 1)
        for c in descs[q]:
            c.wait()

        def group_body(g, carry, q=q):
            b = pl.multiple_of(q * QB, L) + g * L
            hh = half_h[pl.ds(b, L)]
            th = half_t[pl.ds(b, L)]
            res = jnp.zeros((L,), jnp.float32)
            for u in range(L):
                i = g * L + u
                hcb = pl.multiple_of(hh[u], 32)
                tcb = pl.multiple_of(th[u], 32)
                acc = jnp.zeros((L,), jnp.float32)
                for k in range(2):
                    hw = lax.bitcast_convert_type(
                        hbuf[q % 2][i, pl.ds(hcb + k * L, L)], jnp.int32)
                    tw = lax.bitcast_convert_type(
                        tbuf[q % 2][i, pl.ds(tcb + k * L, L)], jnp.int32)
                    for sel in range(2):
                        if sel == 0:
                            hv = lax.bitcast_convert_type(
                                jnp.left_shift(hw, 16), jnp.float32)
                            tv = lax.bitcast_convert_type(
                                jnp.left_shift(tw, 16), jnp.float32)
                        else:
                            hv = lax.bitcast_convert_type(
                                hw & jnp.int32(-65536), jnp.float32)
                            tv = lax.bitcast_convert_type(
                                tw & jnp.int32(-65536), jnp.float32)
                        d = jnp.maximum(tv - hv, 0.0)
                        acc = acc + d * d
                total = jnp.sum(acc)
                res = jnp.where(lanes == u, total, res)
            out_v[pl.ds(b, L)] = _norm_neg_sqrt(res)
            return carry

        lax.fori_loop(0, QB // L, group_body, 0)

    pltpu.sync_copy(out_v, out.at[pl.ds(base, B_PER_W)])


_sc_kernel = functools.partial(
    pl.kernel,
    out_type=jax.ShapeDtypeStruct((BATCH,), jnp.float32),
    mesh=plsc.VectorSubcoreMesh(core_axis_name="c", subcore_axis_name="s"),
    compiler_params=pltpu.CompilerParams(needs_layout_passes=False),
    scratch_types=[
        pltpu.VMEM((B_PER_W,), jnp.int32),
        pltpu.VMEM((B_PER_W,), jnp.int32),
        pltpu.VMEM((B_PER_W,), jnp.int32),
        pltpu.VMEM((B_PER_W,), jnp.int32),
        pltpu.VMEM((B_PER_W,), jnp.int32),
        pltpu.VMEM((B_PER_W,), jnp.int32),
        [pltpu.VMEM((QB, 2 * EMBED_DIM), jnp.float32) for _ in range(2)],
        [pltpu.VMEM((QB, 2 * EMBED_DIM), jnp.float32) for _ in range(2)],
        pltpu.VMEM((B_PER_W,), jnp.float32),
        [pltpu.SemaphoreType.DMA for _ in range(2)],
    ],
)(_sc_body)


def kernel(entity_emb, relation_emb, h, r, t):
    del relation_emb, r
    table_t = entity_emb.T
    packed = _tc_pack(table_t)
    return _sc_kernel(packed, h.astype(jnp.int32), t.astype(jnp.int32))


# speedup vs baseline: 1.0088x; 1.0088x over previous
import functools

import jax
import jax.numpy as jnp
from jax import lax
from jax.experimental import pallas as pl
from jax.experimental.pallas import tpu as pltpu
from jax.experimental.pallas import tpu_sc as plsc

NUM_ENT = 1000000
EMBED_DIM = 64
BATCH = 16384

NC = 2
NS = 16
L = 16
NW = NC * NS
B_PER_W = BATCH // NW
QB = 128
NQ = B_PER_W // QB

TBLK = 32768
HT = TBLK // 2
SHB = TBLK.bit_length() - 1
NBLK = -(-NUM_ENT // TBLK)
QT = TBLK // 4
YROWS = NBLK * QT


def _tc_pack_body(x_ref, y_ref):
    xi = lax.bitcast_convert_type(x_ref[...], jnp.int32)
    lo = jnp.right_shift(xi[0:32, :] + 0x8000, 16) & jnp.int32(0xFFFF)
    hi = (xi[32:64, :] + 0x8000) & jnp.int32(-65536)
    w = lax.bitcast_convert_type(hi | lo, jnp.float32)
    v = jnp.concatenate(
        [w[:, m * QT:(m + 1) * QT] for m in range(4)], axis=0)
    y_ref[...] = v.T


_tc_pack = pl.pallas_call(
    _tc_pack_body,
    grid=(NBLK,),
    in_specs=[pl.BlockSpec((EMBED_DIM, TBLK), lambda i: (0, i))],
    out_specs=pl.BlockSpec((QT, 2 * EMBED_DIM), lambda i: (i, 0)),
    out_shape=jax.ShapeDtypeStruct((YROWS, 2 * EMBED_DIM), jnp.float32),
    compiler_params=pltpu.CompilerParams(
        dimension_semantics=("arbitrary",)),
)


def _norm_neg_sqrt(acc):
    xs = jnp.maximum(acc, 1e-20)
    bits = lax.bitcast_convert_type(xs, jnp.int32)
    bits = jnp.int32(0x5F3759DF) - jnp.right_shift(bits, 1)
    y = lax.bitcast_convert_type(bits, jnp.float32)
    y = y * (1.5 - 0.5 * xs * y * y)
    y = y * (1.5 - 0.5 * xs * y * y)
    return -(xs * y)


def _sc_body(y, h_idx, t_idx, out,
             raw_h, raw_t, row_h, row_t, half_h, half_t,
             hbuf, tbuf, out_v, sems):
    wid = lax.axis_index("s") * NC + lax.axis_index("c")
    base = wid * B_PER_W

    pltpu.sync_copy(h_idx.at[pl.ds(base, B_PER_W)], raw_h)
    pltpu.sync_copy(t_idx.at[pl.ds(base, B_PER_W)], raw_t)

    def prep_body(k, carry):
        off = pl.multiple_of(k * L, L)
        for raw, row, half in ((raw_h, row_h, half_h), (raw_t, row_t, half_t)):
            e = raw[pl.ds(off, L)]
            row[pl.ds(off, L)] = (jnp.left_shift(jnp.right_shift(e, SHB), SHB - 2)
                                  | (e & (QT - 1)))
            half[pl.ds(off, L)] = jnp.left_shift(jnp.right_shift(e, SHB - 2) & 3, 5)
        return carry

    lax.fori_loop(0, B_PER_W // L, prep_body, 0)

    def fire(q):
        return (
            pltpu.async_copy(y.at[row_h.at[pl.ds(q * QB, QB)]],
                             hbuf[q % 2], sems[q % 2]),
            pltpu.async_copy(y.at[row_t.at[pl.ds(q * QB, QB)]],
                             tbuf[q % 2], sems[q % 2]),
        )

    lanes = lax.iota(jnp.int32, L)
    descs = {0: fire(0)}
    for q in range(NQ):
        if q + 1 < NQ:
            descs[q + 1] = fire(q + 1)
        for c in descs[q]:
            c.wait()

        def group_body(g, carry, q=q):
            b = pl.multiple_of(q * QB, L) + g * L
            hh = half_h[pl.ds(b, L)]
            th = half_t[pl.ds(b, L)]
            res = jnp.zeros((L,), jnp.float32)
            for u in range(L):
                i = g * L + u
                hcb = pl.multiple_of(hh[u], 32)
                tcb = pl.multiple_of(th[u], 32)
                acc = jnp.zeros((L,), jnp.float32)
                for k in range(2):
                    hw = lax.bitcast_convert_type(
                        hbuf[q % 2][i, pl.ds(hcb + k * L, L)], jnp.int32)
                    tw = lax.bitcast_convert_type(
                        tbuf[q % 2][i, pl.ds(tcb + k * L, L)], jnp.int32)
                    for sel in range(2):
                        if sel == 0:
                            hv = lax.bitcast_convert_type(
                                jnp.left_shift(hw, 16), jnp.float32)
                            tv = lax.bitcast_convert_type(
                                jnp.left_shift(tw, 16), jnp.float32)
                        else:
                            hv = lax.bitcast_convert_type(
                                hw & jnp.int32(-65536), jnp.float32)
                            tv = lax.bitcast_convert_type(
                                tw & jnp.int32(-65536), jnp.float32)
                        d = jnp.maximum(tv - hv, 0.0)
                        acc = acc + d * d
                total = jnp.sum(acc)
                res = jnp.where(lanes == u, total, res)
            out_v[pl.ds(b, L)] = _norm_neg_sqrt(res)
            return carry

        lax.fori_loop(0, QB // L, group_body, 0)

    pltpu.sync_copy(out_v, out.at[pl.ds(base, B_PER_W)])


_sc_kernel = functools.partial(
    pl.kernel,
    out_type=jax.ShapeDtypeStruct((BATCH,), jnp.float32),
    mesh=plsc.VectorSubcoreMesh(core_axis_name="c", subcore_axis_name="s"),
    compiler_params=pltpu.CompilerParams(needs_layout_passes=False),
    scratch_types=[
        pltpu.VMEM((B_PER_W,), jnp.int32),
        pltpu.VMEM((B_PER_W,), jnp.int32),
        pltpu.VMEM((B_PER_W,), jnp.int32),
        pltpu.VMEM((B_PER_W,), jnp.int32),
        pltpu.VMEM((B_PER_W,), jnp.int32),
        pltpu.VMEM((B_PER_W,), jnp.int32),
        [pltpu.VMEM((QB, 2 * EMBED_DIM), jnp.float32) for _ in range(2)],
        [pltpu.VMEM((QB, 2 * EMBED_DIM), jnp.float32) for _ in range(2)],
        pltpu.VMEM((B_PER_W,), jnp.float32),
        [pltpu.SemaphoreType.DMA for _ in range(2)],
    ],
)(_sc_body)


def kernel(entity_emb, relation_emb, h, r, t):
    del relation_emb, r
    table_t = entity_emb.T
    packed = _tc_pack(table_t)
    return _sc_kernel(packed, h.astype(jnp.int32), t.astype(jnp.int32))
